# baseline (speedup 1.0000x reference)
import functools

import jax
import jax.numpy as jnp
from jax import lax
from jax.experimental import pallas as pl
from jax.experimental.pallas import tpu as pltpu
from jax.experimental.pallas import tpu_sc as plsc

_B, _T, _C = 4, 8192, 768
_NC, _NS = 2, 16
_NW = _NC * _NS
_TPW = _T // _NW
_R = 64
_NTILES = _TPW // _R
_LANES = _C // 16


def _sc_body(x_hbm, pe_hbm, out_hbm, x_buf, pe_buf):
    wid = lax.axis_index("s") * _NC + lax.axis_index("c")
    t0 = wid * _TPW
    for tile in range(_NTILES):
        trow = t0 + tile * _R
        pltpu.sync_copy(pe_hbm.at[pl.ds(trow, _R), :], pe_buf)
        for b in range(_B):
            row = b * _T + trow
            pltpu.sync_copy(x_hbm.at[pl.ds(row, _R), :], x_buf)

            def _add_row(r, carry):
                for c in range(_LANES):
                    sl = pl.ds(c * 16, 16)
                    x_buf[r, sl] = x_buf[r, sl] + pe_buf[r, sl]
                return carry

            lax.fori_loop(0, _R, _add_row, 0)
            pltpu.sync_copy(x_buf, out_hbm.at[pl.ds(row, _R), :])


def kernel(x, pe_table):
    B, T, C = x.shape
    x2 = x.reshape(B * T, C)
    sc_add = functools.partial(
        pl.kernel,
        mesh=plsc.VectorSubcoreMesh(core_axis_name="c", subcore_axis_name="s"),
        out_type=jax.ShapeDtypeStruct((B * T, C), jnp.float32),
        scratch_types=[
            pltpu.VMEM((_R, C), jnp.float32),
            pltpu.VMEM((_R, C), jnp.float32),
        ],
    )(_sc_body)
    out = sc_add(x2, pe_table[:T])
    return out.reshape(B, T, C)


# speedup vs baseline: 1.5562x; 1.5562x over previous
import functools

import jax
import jax.numpy as jnp
from jax import lax
from jax.experimental import pallas as pl
from jax.experimental.pallas import tpu as pltpu
from jax.experimental.pallas import tpu_sc as plsc

_B, _T, _C = 4, 8192, 768
_TS = 2048
_NC, _NS = 2, 16
_NW = _NC * _NS
_TPW = _TS // _NW
_R = 64
_NTILES = _TPW // _R
_LANES = _C // 16
_BT = 2048


def _sc_body(x_hbm, pe_hbm, out_hbm, x_buf, pe_buf):
    wid = lax.axis_index("s") * _NC + lax.axis_index("c")
    t0 = wid * _TPW
    for tile in range(_NTILES):
        trow = t0 + tile * _R
        pltpu.sync_copy(pe_hbm.at[pl.ds(trow, _R), :], pe_buf)
        for b in range(_B):
            row = b * _T + trow
            out_row = b * _TS + trow
            pltpu.sync_copy(x_hbm.at[pl.ds(row, _R), :], x_buf)

            def _add_row(r, carry):
                for c in range(_LANES):
                    sl = pl.ds(c * 16, 16)
                    x_buf[r, sl] = x_buf[r, sl] + pe_buf[r, sl]
                return carry

            lax.fori_loop(0, _R, _add_row, 0)
            pltpu.sync_copy(x_buf, out_hbm.at[pl.ds(out_row, _R), :])


def _tc_body(x_ref, pe_ref, o_ref):
    o_ref[...] = x_ref[...] + pe_ref[...][None, :, :]


def kernel(x, pe_table):
    B, T, C = x.shape
    nt = (T - _TS) // _BT
    toff = _TS // _BT
    tc_full = pl.pallas_call(
        _tc_body,
        grid=(nt, B),
        in_specs=[
            pl.BlockSpec((1, _BT, C), lambda t, b: (b, t + toff, 0)),
            pl.BlockSpec((_BT, C), lambda t, b: (t + toff, 0)),
        ],
        out_specs=pl.BlockSpec((1, _BT, C), lambda t, b: (b, t + toff, 0)),
        out_shape=jax.ShapeDtypeStruct((B, T, C), x.dtype),
    )(x, pe_table[:T])

    sc_add = functools.partial(
        pl.kernel,
        mesh=plsc.VectorSubcoreMesh(core_axis_name="c", subcore_axis_name="s"),
        out_type=jax.ShapeDtypeStruct((B * _TS, C), jnp.float32),
        scratch_types=[
            pltpu.VMEM((_R, C), jnp.float32),
            pltpu.VMEM((_R, C), jnp.float32),
        ],
    )(_sc_body)
    sc_out = sc_add(x.reshape(B * T, C), pe_table[:_TS])

    return lax.dynamic_update_slice(
        tc_full, sc_out.reshape(B, _TS, C), (0, 0, 0)
    )
